# baseline (speedup 1.0000x reference)
import functools

import jax
import jax.numpy as jnp
from jax import lax
from jax.experimental import pallas as pl
from jax.experimental.pallas import tpu as pltpu
from jax.experimental.pallas import tpu_sc as plsc

_VOCAB = 33
_DIM = 1280
_BATCH = 4
_SEQ = 2048
_B_TOTAL = _BATCH * _SEQ
_NUM_WORKERS = 32
_B_PER_W = _B_TOTAL // _NUM_WORKERS
_W_PER_ROW = _SEQ // _B_PER_W
_LANES = 16
_NCHUNK = _B_PER_W // _LANES


def _body(table_hbm, idx_hbm, out_hbm, table_v, idx_v,
          sem0, sem1, sem2, sem3):
    wid = lax.axis_index("s") * 2 + lax.axis_index("c")
    base = wid * _B_PER_W
    sems = (sem0, sem1, sem2, sem3)

    pltpu.sync_copy(table_hbm, table_v)
    pltpu.sync_copy(
        idx_hbm.at[wid // _W_PER_ROW,
                   pl.ds((wid % _W_PER_ROW) * _B_PER_W, _B_PER_W)],
        idx_v)

    def fire(c, sem):
        ids = idx_v[pl.ds(c * _LANES, _LANES)]
        for lane in range(_LANES):
            pltpu.async_copy(
                table_v.at[ids[lane]],
                out_hbm.at[base + c * _LANES + lane],
                sem)

    def drain(sem):
        for _ in range(_LANES):
            pltpu.make_async_copy(
                table_v.at[0], out_hbm.at[0], sem).wait()

    def quad(p, _):
        for k in range(4):
            @pl.when(p > 0)
            def _():
                drain(sems[k])
            fire(4 * p + k, sems[k])
        return 0

    lax.fori_loop(0, _NCHUNK // 4, quad, 0)
    for k in range(4):
        drain(sems[k])


_gather = functools.partial(
    pl.kernel,
    out_type=jax.ShapeDtypeStruct((_B_TOTAL, _DIM), jnp.float32),
    mesh=plsc.VectorSubcoreMesh(core_axis_name="c", subcore_axis_name="s"),
    scratch_types=[
        pltpu.VMEM((_VOCAB, _DIM), jnp.float32),
        pltpu.VMEM((_B_PER_W,), jnp.int32),
        pltpu.SemaphoreType.DMA,
        pltpu.SemaphoreType.DMA,
        pltpu.SemaphoreType.DMA,
        pltpu.SemaphoreType.DMA,
    ],
)(_body)


@jax.jit
def kernel(input_ids, weight):
    out = _gather(weight.astype(jnp.float32), input_ids.astype(jnp.int32))
    return out.reshape(_BATCH, _SEQ, _DIM).astype(weight.dtype)


# speedup vs baseline: 1.0350x; 1.0350x over previous
import functools

import jax
import jax.numpy as jnp
from jax import lax
from jax.experimental import pallas as pl
from jax.experimental.pallas import tpu as pltpu
from jax.experimental.pallas import tpu_sc as plsc

_VOCAB = 33
_DIM = 1280
_BATCH = 4
_SEQ = 2048
_B_TOTAL = _BATCH * _SEQ
_NUM_WORKERS = 32
_B_PER_W = _B_TOTAL // _NUM_WORKERS
_W_PER_ROW = _SEQ // _B_PER_W
_LANES = 16
_NCHUNK = _B_PER_W // _LANES


def _body(table_hbm, idx_hbm, out_hbm, table_v, idx_v, sem0, sem1):
    wid = lax.axis_index("s") * 2 + lax.axis_index("c")
    base = wid * _B_PER_W
    sems = (sem0, sem1)

    pltpu.sync_copy(table_hbm, table_v)
    pltpu.sync_copy(
        idx_hbm.at[wid // _W_PER_ROW,
                   pl.ds((wid % _W_PER_ROW) * _B_PER_W, _B_PER_W)],
        idx_v)

    def fire(c, sem):
        ids = idx_v[pl.ds(c * _LANES, _LANES)]
        for lane in range(_LANES):
            pltpu.async_copy(
                table_v.at[ids[lane]],
                out_hbm.at[base + c * _LANES + lane],
                sem)

    def drain(sem):
        for _ in range(_LANES):
            pltpu.make_async_copy(
                table_v.at[0], out_hbm.at[0], sem).wait()

    def pair(p, _):
        @pl.when(p > 0)
        def _():
            drain(sems[0])
        fire(2 * p, sems[0])

        @pl.when(p > 0)
        def _():
            drain(sems[1])
        fire(2 * p + 1, sems[1])
        return 0

    lax.fori_loop(0, _NCHUNK // 2, pair, 0)
    drain(sems[0])
    drain(sems[1])


_gather = functools.partial(
    pl.kernel,
    out_type=jax.ShapeDtypeStruct((_B_TOTAL, _DIM), jnp.float32),
    mesh=plsc.VectorSubcoreMesh(core_axis_name="c", subcore_axis_name="s"),
    scratch_types=[
        pltpu.VMEM((_VOCAB, _DIM), jnp.float32),
        pltpu.VMEM((_B_PER_W,), jnp.int32),
        pltpu.SemaphoreType.DMA,
        pltpu.SemaphoreType.DMA,
    ],
)(_body)


@jax.jit
def kernel(input_ids, weight):
    out = _gather(weight.astype(jnp.float32), input_ids.astype(jnp.int32))
    return out.reshape(_BATCH, _SEQ, _DIM).astype(weight.dtype)
